# baseline (speedup 1.0000x reference)
import functools

import jax
import jax.numpy as jnp
from jax.experimental import pallas as pl

N = 4096
B = 8
N_POINTS = 1024
N_SAMPLES = 32
RADIUS = 0.2
BN_EPS = 1e-3


def _fps_body(x_ref, y_ref, z_ref, first_ref, cid_ref, cx_ref, cy_ref, cz_ref):
    X = x_ref[...]
    Y = y_ref[...]
    Z = z_ref[...]
    lane = jax.lax.broadcasted_iota(jnp.int32, (B, N), 1)

    def extract(oh, V):
        return jnp.sum(jnp.where(oh, V, 0.0), axis=1, keepdims=True)

    idx0 = first_ref[...]
    oh0 = lane == idx0
    px, py, pz = extract(oh0, X), extract(oh0, Y), extract(oh0, Z)
    lane128 = jax.lax.broadcasted_iota(jnp.int32, (B, 128), 1)

    def body(p, carry):
        px, py, pz, a_id, a_x, a_y, a_z, base = carry
        dx = X - px
        dy = Y - py
        dz = Z - pz
        d2 = dx * dx + dy * dy + dz * dz
        dist = jnp.sqrt(d2 + 1e-12)
        m = jnp.max(dist, axis=1, keepdims=True)
        idx = jnp.min(jnp.where(dist == m, lane, N), axis=1, keepdims=True)
        oh = lane == idx
        npx, npy, npz = extract(oh, X), extract(oh, Y), extract(oh, Z)
        sel = lane128 == (p - base)
        a_id = jnp.where(sel, idx, a_id)
        a_x = jnp.where(sel, npx, a_x)
        a_y = jnp.where(sel, npy, a_y)
        a_z = jnp.where(sel, npz, a_z)
        return (npx, npy, npz, a_id, a_x, a_y, a_z, base)

    zf = jnp.zeros((B, 128), jnp.float32)
    zi = jnp.zeros((B, 128), jnp.int32)
    sel0 = lane128 == 0
    a_id = jnp.where(sel0, idx0, zi)
    a_x = jnp.where(sel0, px, zf)
    a_y = jnp.where(sel0, py, zf)
    a_z = jnp.where(sel0, pz, zf)
    carry = (px, py, pz, a_id, a_x, a_y, a_z, jnp.int32(0))
    for c in range(N_POINTS // 128):
        base = c * 128
        lo = 1 if c == 0 else base
        px, py, pz, a_id, a_x, a_y, a_z, _ = carry
        carry = jax.lax.fori_loop(
            lo, base + 128, body,
            (px, py, pz, a_id, a_x, a_y, a_z, jnp.int32(base)))
        px, py, pz, a_id, a_x, a_y, a_z, _ = carry
        cid_ref[:, base:base + 128] = a_id
        cx_ref[:, base:base + 128] = a_x
        cy_ref[:, base:base + 128] = a_y
        cz_ref[:, base:base + 128] = a_z
        carry = (px, py, pz, zi, zf, zf, zf, jnp.int32(0))


def _fps_pallas(xs, ys, zs, first, interpret=False):
    outs = jax.ShapeDtypeStruct((B, N_POINTS), jnp.int32)
    outf = jax.ShapeDtypeStruct((B, N_POINTS), jnp.float32)
    return pl.pallas_call(
        _fps_body,
        out_shape=(outs, outf, outf, outf),
        interpret=interpret,
    )(xs, ys, zs, first)


def _query_ball_point(radius, n_samples, xyz, cent_xyz):
    x2 = jnp.sum(xyz ** 2, axis=2)
    c2 = jnp.sum(cent_xyz ** 2, axis=2)
    xc = jnp.einsum('bnd,bmd->bnm', cent_xyz, xyz)
    d2 = jnp.maximum(c2[:, :, None] + x2[:, None, :] - 2.0 * xc, 0.0)
    dist = jnp.sqrt(d2 + 1e-12)
    dist = jnp.minimum(dist, radius ** 2)
    return jnp.argsort(dist, axis=2)[:, :, :n_samples]


def _mlp_bn_relu(x, W, b, gamma, beta):
    x = jnp.einsum('...d,df->...f', x, W) + b
    mean = jnp.mean(x, axis=(0, 1, 2), keepdims=True)
    var = jnp.var(x, axis=(0, 1, 2), keepdims=True)
    x = gamma * (x - mean) * jax.lax.rsqrt(var + BN_EPS) + beta
    return jax.nn.relu(x)


def kernel(inputs, W0, b0, gamma0, beta0, W1, b1, gamma1, beta1, W2, b2, gamma2, beta2):
    key = jax.random.key(42)
    first = jax.random.randint(key, (B,), 0, N - 1, dtype=jnp.int32)
    xs = inputs[:, :, 0]
    ys = inputs[:, :, 1]
    zs = inputs[:, :, 2]
    cids, cx, cy, cz = _fps_pallas(xs, ys, zs, first[:, None])
    cent_xyz = jnp.stack([cx, cy, cz], axis=-1)

    group_idx = _query_ball_point(RADIUS, N_SAMPLES, inputs, cent_xyz)
    group_xyz = jnp.take_along_axis(inputs[:, None, :, :], group_idx[..., None], axis=2)
    x = group_xyz
    for (W, b, g, bt) in ((W0, b0, gamma0, beta0), (W1, b1, gamma1, beta1), (W2, b2, gamma2, beta2)):
        x = _mlp_bn_relu(x, W, b, g, bt)
    return jnp.max(x, axis=2)


# speedup vs baseline: 8.6348x; 8.6348x over previous
import functools

import jax
import jax.numpy as jnp
from jax import lax
from jax.experimental import pallas as pl
from jax.experimental.pallas import tpu as pltpu
from jax.experimental.pallas import tpu_sc as plsc

N = 4096
B = 8
N_POINTS = 1024
N_SAMPLES = 32
RADIUS = 0.2
BN_EPS = 1e-3


def _fps_body(x_ref, y_ref, z_ref, first_ref, cid_ref, cx_ref, cy_ref, cz_ref):
    X = x_ref[...]
    Y = y_ref[...]
    Z = z_ref[...]
    lane = jax.lax.broadcasted_iota(jnp.int32, (B, N), 1)

    def extract(oh, V):
        return jnp.sum(jnp.where(oh, V, 0.0), axis=1, keepdims=True)

    idx0 = first_ref[...]
    oh0 = lane == idx0
    px, py, pz = extract(oh0, X), extract(oh0, Y), extract(oh0, Z)
    lane128 = jax.lax.broadcasted_iota(jnp.int32, (B, 128), 1)

    def body(p, carry):
        px, py, pz, a_id, a_x, a_y, a_z, base = carry
        dx = X - px
        dy = Y - py
        dz = Z - pz
        d2 = dx * dx + dy * dy + dz * dz
        dist = jnp.sqrt(d2 + 1e-12)
        m = jnp.max(dist, axis=1, keepdims=True)
        idx = jnp.min(jnp.where(dist == m, lane, N), axis=1, keepdims=True)
        oh = lane == idx
        npx, npy, npz = extract(oh, X), extract(oh, Y), extract(oh, Z)
        sel = lane128 == (p - base)
        a_id = jnp.where(sel, idx, a_id)
        a_x = jnp.where(sel, npx, a_x)
        a_y = jnp.where(sel, npy, a_y)
        a_z = jnp.where(sel, npz, a_z)
        return (npx, npy, npz, a_id, a_x, a_y, a_z, base)

    zf = jnp.zeros((B, 128), jnp.float32)
    zi = jnp.zeros((B, 128), jnp.int32)
    sel0 = lane128 == 0
    a_id = jnp.where(sel0, idx0, zi)
    a_x = jnp.where(sel0, px, zf)
    a_y = jnp.where(sel0, py, zf)
    a_z = jnp.where(sel0, pz, zf)
    carry = (px, py, pz, a_id, a_x, a_y, a_z, jnp.int32(0))
    for c in range(N_POINTS // 128):
        base = c * 128
        lo = 1 if c == 0 else base
        px, py, pz, a_id, a_x, a_y, a_z, _ = carry
        carry = jax.lax.fori_loop(
            lo, base + 128, body,
            (px, py, pz, a_id, a_x, a_y, a_z, jnp.int32(base)))
        px, py, pz, a_id, a_x, a_y, a_z, _ = carry
        cid_ref[:, base:base + 128] = a_id
        cx_ref[:, base:base + 128] = a_x
        cy_ref[:, base:base + 128] = a_y
        cz_ref[:, base:base + 128] = a_z
        carry = (px, py, pz, zi, zf, zf, zf, jnp.int32(0))


def _fps_pallas(xs, ys, zs, first, interpret=False):
    outs = jax.ShapeDtypeStruct((B, N_POINTS), jnp.int32)
    outf = jax.ShapeDtypeStruct((B, N_POINTS), jnp.float32)
    return pl.pallas_call(
        _fps_body,
        out_shape=(outs, outf, outf, outf),
        interpret=interpret,
    )(xs, ys, zs, first)


import numpy as _np
C_CLAMP = _np.float32(RADIUS ** 2)


def _dist_body(cp_ref, xt_ref, d_ref):
    cp = cp_ref[0]
    xt = xt_ref[0]
    c2 = jnp.sum(cp * cp, axis=1, keepdims=True)
    x2 = jnp.sum(xt * xt, axis=0, keepdims=True)
    xc = lax.dot_general(
        cp.astype(jnp.bfloat16), xt.astype(jnp.bfloat16),
        (((1,), (0,)), ((), ())), preferred_element_type=jnp.float32)
    d2 = jnp.maximum((c2 + x2) - 2.0 * xc, 0.0)
    dist = jnp.sqrt(d2 + 1e-12)
    d_ref[0] = jnp.minimum(dist, C_CLAMP)


def _dist_pallas(cpad, xt):
    return pl.pallas_call(
        _dist_body,
        grid=(B, N_POINTS // 128),
        in_specs=[
            pl.BlockSpec((1, 128, 8), lambda b, t: (b, t, 0)),
            pl.BlockSpec((1, 8, N), lambda b, t: (b, 0, 0)),
        ],
        out_specs=pl.BlockSpec((1, 128, N), lambda b, t: (b, t, 0)),
        out_shape=jax.ShapeDtypeStruct((B, N_POINTS, N), jnp.float32),
    )(cpad, xt)


_NW = 32
_RPW = (B * N_POINTS) // _NW
_I16 = None


def _sc_select_body(d_hbm, xs_hbm, ys_hbm, zs_hbm,
                    gx_hbm, gy_hbm, gz_hbm,
                    rowbuf, xs_v, ys_v, zs_v,
                    selbuf, idxbuf, valbuf,
                    outx, outy, outz, sem):
    w = lax.axis_index("s") * 2 + lax.axis_index("c")
    r0 = w * _RPW
    bb = r0 // N_POINTS
    iota16 = lax.broadcasted_iota(jnp.int32, (16,), 0)

    pltpu.sync_copy(xs_hbm.at[bb], xs_v)
    pltpu.sync_copy(ys_hbm.at[bb], ys_v)
    pltpu.sync_copy(zs_hbm.at[bb], zs_v)

    pltpu.async_copy(d_hbm.at[r0], rowbuf.at[0], sem)

    def row_step(rl, carry):
        cur = lax.rem(rl, 2)
        pltpu.make_async_copy(d_hbm.at[r0 + rl], rowbuf.at[cur], sem).wait()

        @pl.when(rl < _RPW - 1)
        def _():
            pltpu.async_copy(d_hbm.at[r0 + rl + 1], rowbuf.at[1 - cur], sem)

        def chunk(ci, q):
            d = rowbuf[cur, pl.ds(ci * 16, 16)]
            m = d < C_CLAMP
            iv = ci * 16 + iota16
            plsc.store_compressed(idxbuf.at[pl.ds(q, 16)], iv, mask=m)
            plsc.store_compressed(valbuf.at[pl.ds(q, 16)], d, mask=m)
            return q + jnp.sum(jnp.where(m, 1, 0))

        q = lax.fori_loop(0, N // 16, chunk, jnp.int32(0))

        def fill(_):
            cnt = q
            rank_base = jnp.int32(0)
            for j in range(2):
                d = rowbuf[cur, pl.ds(j * 16, 16)]
                far = d >= C_CLAMP
                fr = jnp.where(far, 1, 0)
                inc = plsc.cumsum(fr)
                rank = rank_base + (inc - fr)
                sel = far & (rank < 32 - q)
                iv = j * 16 + iota16
                plsc.store_compressed(idxbuf.at[pl.ds(cnt, 16)], iv, mask=sel)
                cnt = cnt + jnp.sum(jnp.where(sel, 1, 0))
                rank_base = rank_base + jnp.sum(fr)
            for j in range(2):
                selbuf[pl.ds(j * 16, 16)] = idxbuf[pl.ds(j * 16, 16)]
            return jnp.int32(0)

        def topk(_):
            nch = (q + 15) // 16
            big_f = jnp.float32(3.0e38)
            big_i = jnp.int32(2 ** 30)

            def pick(k, sels):
                s0, s1 = sels

                def scan(ci, st):
                    bv, bi, bp = st
                    p = ci * 16 + iota16
                    valid = p < q
                    v = jnp.where(valid, valbuf[pl.ds(ci * 16, 16)], big_f)
                    i = jnp.where(valid, idxbuf[pl.ds(ci * 16, 16)], big_i)
                    lt = (v < bv) | ((v == bv) & (i < bi))
                    return (jnp.where(lt, v, bv), jnp.where(lt, i, bi),
                            jnp.where(lt, p, bp))

                bv, bi, bp = lax.fori_loop(
                    0, nch, scan,
                    (jnp.full((16,), big_f), jnp.full((16,), big_i),
                     jnp.full((16,), big_i)))
                mv = jnp.min(bv)
                cand = bv == mv
                mi = jnp.min(jnp.where(cand, bi, big_i))
                pos = jnp.min(jnp.where(cand & (bi == mi), bp, big_i))
                s0 = jnp.where(iota16 == k, mi, s0)
                s1 = jnp.where(iota16 == (k - 16), mi, s1)
                cb = (pos // 16) * 16
                lanepos = pos - cb
                v = valbuf[pl.ds(cb, 16)]
                valbuf[pl.ds(cb, 16)] = jnp.where(iota16 == lanepos, big_f, v)
                return (s0, s1)

            s0, s1 = lax.fori_loop(
                0, 32, pick,
                (jnp.zeros((16,), jnp.int32), jnp.zeros((16,), jnp.int32)))
            selbuf[pl.ds(0, 16)] = s0
            selbuf[pl.ds(16, 16)] = s1
            return jnp.int32(0)

        lax.cond(q < 32, fill, topk, jnp.int32(0))

        for j in range(2):
            si = selbuf[pl.ds(j * 16, 16)]
            outx[rl, pl.ds(j * 16, 16)] = plsc.load_gather(xs_v, [si])
            outy[rl, pl.ds(j * 16, 16)] = plsc.load_gather(ys_v, [si])
            outz[rl, pl.ds(j * 16, 16)] = plsc.load_gather(zs_v, [si])
        return carry

    lax.fori_loop(0, _RPW, row_step, jnp.int32(0))

    pltpu.sync_copy(outx, gx_hbm.at[pl.ds(r0, _RPW)])
    pltpu.sync_copy(outy, gy_hbm.at[pl.ds(r0, _RPW)])
    pltpu.sync_copy(outz, gz_hbm.at[pl.ds(r0, _RPW)])


def _sc_select(drows, xs, ys, zs):
    og = jax.ShapeDtypeStruct((B * N_POINTS, N_SAMPLES), jnp.float32)
    mesh = plsc.VectorSubcoreMesh(core_axis_name="c", subcore_axis_name="s")
    f = pl.kernel(
        _sc_select_body,
        mesh=mesh,
        compiler_params=pltpu.CompilerParams(needs_layout_passes=False),
        out_type=(og, og, og),
        scratch_types=[
            pltpu.VMEM((2, N), jnp.float32),
            pltpu.VMEM((N,), jnp.float32),
            pltpu.VMEM((N,), jnp.float32),
            pltpu.VMEM((N,), jnp.float32),
            pltpu.VMEM((32,), jnp.int32),
            pltpu.VMEM((N + 32,), jnp.int32),
            pltpu.VMEM((N + 32,), jnp.float32),
            pltpu.VMEM((_RPW, N_SAMPLES), jnp.float32),
            pltpu.VMEM((_RPW, N_SAMPLES), jnp.float32),
            pltpu.VMEM((_RPW, N_SAMPLES), jnp.float32),
            pltpu.SemaphoreType.DMA,
        ],
    )
    return f(drows, xs, ys, zs)


def _mlp_bn_relu(x, W, b, gamma, beta):
    x = jnp.einsum('...d,df->...f', x, W) + b
    mean = jnp.mean(x, axis=(0, 1, 2), keepdims=True)
    var = jnp.var(x, axis=(0, 1, 2), keepdims=True)
    x = gamma * (x - mean) * jax.lax.rsqrt(var + BN_EPS) + beta
    return jax.nn.relu(x)


def kernel(inputs, W0, b0, gamma0, beta0, W1, b1, gamma1, beta1, W2, b2, gamma2, beta2):
    key = jax.random.key(42)
    first = jax.random.randint(key, (B,), 0, N - 1, dtype=jnp.int32)
    xs = inputs[:, :, 0]
    ys = inputs[:, :, 1]
    zs = inputs[:, :, 2]
    cids, cx, cy, cz = _fps_pallas(xs, ys, zs, first[:, None])

    cpad = jnp.pad(jnp.stack([cx, cy, cz], axis=-1), ((0, 0), (0, 0), (0, 5)))
    xt = jnp.pad(jnp.stack([xs, ys, zs], axis=1), ((0, 0), (0, 5), (0, 0)))
    drows = _dist_pallas(cpad, xt).reshape(B * N_POINTS, N)
    gx, gy, gz = _sc_select(drows, xs, ys, zs)
    group_xyz = jnp.stack([gx, gy, gz], axis=-1).reshape(B, N_POINTS, N_SAMPLES, 3)
    x = group_xyz
    for (W, b, g, bt) in ((W0, b0, gamma0, beta0), (W1, b1, gamma1, beta1), (W2, b2, gamma2, beta2)):
        x = _mlp_bn_relu(x, W, b, g, bt)
    return jnp.max(x, axis=2)
